# baseline (speedup 1.0000x reference)
import functools

import jax
import jax.numpy as jnp
from jax import lax
from jax.experimental import pallas as pl
from jax.experimental.pallas import tpu as pltpu
from jax.experimental.pallas import tpu_sc as plsc

_N = 10000
_E = 320000
_DH = 32
_NP = 10112

_NC = 2
_NS = 16
_NW = _NC * _NS
_BATCH = 128
_K = 80
_EPAD = _NW * _BATCH * _K
_RPT = _NP // _NS
_NBUF = 8
_LOOK = 4


_mesh = plsc.VectorSubcoreMesh(core_axis_name="c", subcore_axis_name="s")


@functools.partial(
    pl.kernel,
    out_type=jax.ShapeDtypeStruct((_NC * _NP, _DH), jnp.float32),
    mesh=_mesh,
    scratch_types=[
        pltpu.VMEM((_K, _BATCH), jnp.int32),
        pltpu.VMEM((_K, _BATCH), jnp.int32),
        pltpu.VMEM((_NBUF, _BATCH, _DH), jnp.float32),
        pltpu.VMEM_SHARED((_NP, _DH), jnp.float32),
        pltpu.SemaphoreType.DMA((_NBUF,)),
        pltpu.SemaphoreType.DMA((_NBUF,)),
    ],
    compiler_params=pltpu.CompilerParams(use_tc_tiling_on_sc=False),
)
def _sc_agg(y_hbm, src_hbm, dst_hbm, zeros_hbm, out_hbm,
            src_v, dst_v, rows_v, agg_sh, gsem, ssem):
    cid = lax.axis_index("c")
    sid = lax.axis_index("s")
    wid = cid * _NS + sid
    rbase = sid * _RPT

    pltpu.sync_copy(zeros_hbm, agg_sh.at[pl.ds(rbase, _RPT)])
    pltpu.sync_copy(src_hbm.at[wid], src_v)
    pltpu.sync_copy(dst_hbm.at[wid], dst_v)
    plsc.subcore_barrier()

    def gath(k, b):
        pltpu.async_copy(y_hbm.at[src_v.at[k]], rows_v.at[b], gsem.at[b])

    def gath_wait(k, b):
        pltpu.make_async_copy(y_hbm.at[src_v.at[k]], rows_v.at[b],
                              gsem.at[b]).wait()

    def scat(k, b):
        pltpu.async_copy(rows_v.at[b], agg_sh.at[dst_v.at[k]],
                         ssem.at[b], add=True)

    def scat_wait(k, b):
        pltpu.make_async_copy(rows_v.at[b], agg_sh.at[dst_v.at[k]],
                              ssem.at[b]).wait()

    for j in range(_LOOK):
        gath(j, j)

    def body(kk, _):
        k0 = kk * _NBUF
        for b in range(_NBUF):
            k = k0 + b
            nb = (b + _LOOK) % _NBUF

            @pl.when(k >= _NBUF - _LOOK)
            def _():
                scat_wait(k - (_NBUF - _LOOK), nb)

            @pl.when(k + _LOOK < _K)
            def _():
                gath(k + _LOOK, nb)

            gath_wait(k, b)
            scat(k, b)
        return 0

    lax.fori_loop(0, _K // _NBUF, body, 0)
    for j in range(_K - (_NBUF - _LOOK), _K):
        scat_wait(j, j % _NBUF)
    plsc.subcore_barrier()

    pltpu.sync_copy(agg_sh.at[pl.ds(rbase, _RPT)],
                    out_hbm.at[pl.ds(cid * _NP + rbase, _RPT)])


def _tc_in_body(h_ref, w1_ref, y_ref):
    y_ref[...] = jnp.dot(h_ref[...], w1_ref[...],
                         preferred_element_type=jnp.float32)


def _tc_mid_body(y_ref, agg_ref, b1_ref, w2_ref, b2_ref, w1n_ref,
                 yn_ref, s_ref):
    z = y_ref[...] + agg_ref[0] + agg_ref[1] + b1_ref[...]
    z = jnp.maximum(z, 0.0)
    xn = jnp.dot(z, w2_ref[...], preferred_element_type=jnp.float32)
    xn = xn + b2_ref[...]
    mask = lax.broadcasted_iota(jnp.int32, xn.shape, 0) < _N
    xn = jnp.where(mask, xn, 0.0)
    yn_ref[...] = jnp.dot(xn, w1n_ref[...], preferred_element_type=jnp.float32)
    s_ref[...] = jnp.sum(xn, axis=0, keepdims=True) * (1.0 / _N)


def _tc_out_body(y_ref, agg_ref, b1_ref, w2_ref, b2_ref, s_ref):
    z = y_ref[...] + agg_ref[0] + agg_ref[1] + b1_ref[...]
    z = jnp.maximum(z, 0.0)
    xn = jnp.dot(z, w2_ref[...], preferred_element_type=jnp.float32)
    xn = xn + b2_ref[...]
    mask = lax.broadcasted_iota(jnp.int32, xn.shape, 0) < _N
    xn = jnp.where(mask, xn, 0.0)
    s_ref[...] = jnp.sum(xn, axis=0, keepdims=True) * (1.0 / _N)


_tc_in = pl.pallas_call(
    _tc_in_body,
    out_shape=jax.ShapeDtypeStruct((_NP, _DH), jnp.float32),
)

_tc_mid = pl.pallas_call(
    _tc_mid_body,
    out_shape=(
        jax.ShapeDtypeStruct((_NP, _DH), jnp.float32),
        jax.ShapeDtypeStruct((1, _DH), jnp.float32),
    ),
)

_tc_out = pl.pallas_call(
    _tc_out_body,
    out_shape=jax.ShapeDtypeStruct((1, _DH), jnp.float32),
)


def kernel(h, edge_index, W1_0, b1_0, W2_0, b2_0, W1_1, b1_1, W2_1, b2_1,
           W1_2, b1_2, W2_2, b2_2):
    src = edge_index[0]
    dst = edge_index[1]
    pad = _EPAD - _E
    srcp = jnp.concatenate(
        [src, jnp.full((pad,), _N, jnp.int32)]).reshape(_NW, _K, _BATCH)
    dstp = jnp.concatenate(
        [dst, jnp.zeros((pad,), jnp.int32)]).reshape(_NW, _K, _BATCH)
    h_pad = jnp.pad(h, ((0, _NP - _N), (0, 0)))
    zeros = jnp.zeros((_RPT, _DH), jnp.float32)

    y = _tc_in(h_pad, W1_0)

    agg = _sc_agg(y, srcp, dstp, zeros).reshape(_NC, _NP, _DH)
    y, s0 = _tc_mid(y, agg, b1_0.reshape(1, _DH), W2_0,
                    b2_0.reshape(1, _DH), W1_1)

    agg = _sc_agg(y, srcp, dstp, zeros).reshape(_NC, _NP, _DH)
    y, s1 = _tc_mid(y, agg, b1_1.reshape(1, _DH), W2_1,
                    b2_1.reshape(1, _DH), W1_2)

    agg = _sc_agg(y, srcp, dstp, zeros).reshape(_NC, _NP, _DH)
    s2 = _tc_out(y, agg, b1_2.reshape(1, _DH), W2_2, b2_2.reshape(1, _DH))

    return jnp.concatenate([s0[0], s1[0], s2[0]])


# speedup vs baseline: 2.2442x; 2.2442x over previous
import functools

import jax
import jax.numpy as jnp
from jax import lax
from jax.experimental import pallas as pl
from jax.experimental.pallas import tpu as pltpu
from jax.experimental.pallas import tpu_sc as plsc

_N = 10000
_E = 320000
_DH = 32
_NP = 10112

_NC = 2
_NS = 16
_NW = _NC * _NS
_BATCH = 128
_K = 80
_EPAD = _NW * _BATCH * _K
_RPT = _NP // _NS
_NBUF = 8
_LOOK = 4


_mesh = plsc.VectorSubcoreMesh(core_axis_name="c", subcore_axis_name="s")


@functools.partial(
    pl.kernel,
    out_type=jax.ShapeDtypeStruct((_NC * _NP, _DH), jnp.float32),
    mesh=_mesh,
    scratch_types=[
        pltpu.VMEM((_K, _BATCH), jnp.int32),
        pltpu.VMEM((_K, _BATCH), jnp.int32),
        pltpu.VMEM((_NBUF, _BATCH, _DH), jnp.float32),
        pltpu.VMEM_SHARED((_NP, _DH), jnp.float32),
        pltpu.VMEM_SHARED((_NP, _DH), jnp.float32),
        pltpu.SemaphoreType.DMA((_NBUF,)),
        pltpu.SemaphoreType.DMA((_NBUF,)),
    ],
    compiler_params=pltpu.CompilerParams(use_tc_tiling_on_sc=False),
)
def _sc_agg(y_hbm, src_hbm, dst_hbm, zeros_hbm, out_hbm,
            src_v, dst_v, rows_v, agg_sh, tbl_sh, gsem, ssem):
    cid = lax.axis_index("c")
    sid = lax.axis_index("s")
    wid = cid * _NS + sid
    rbase = sid * _RPT

    pltpu.sync_copy(y_hbm.at[pl.ds(rbase, _RPT)],
                    tbl_sh.at[pl.ds(rbase, _RPT)])
    pltpu.sync_copy(zeros_hbm, agg_sh.at[pl.ds(rbase, _RPT)])
    pltpu.sync_copy(src_hbm.at[wid], src_v)
    pltpu.sync_copy(dst_hbm.at[wid], dst_v)
    plsc.subcore_barrier()

    def gath(k, b):
        pltpu.async_copy(tbl_sh.at[src_v.at[k]], rows_v.at[b], gsem.at[b])

    def gath_wait(k, b):
        pltpu.make_async_copy(tbl_sh.at[src_v.at[k]], rows_v.at[b],
                              gsem.at[b]).wait()

    def scat(k, b):
        pltpu.async_copy(rows_v.at[b], agg_sh.at[dst_v.at[k]],
                         ssem.at[b], add=True)

    def scat_wait(k, b):
        pltpu.make_async_copy(rows_v.at[b], agg_sh.at[dst_v.at[k]],
                              ssem.at[b]).wait()

    for j in range(_LOOK):
        gath(j, j)

    def body(kk, _):
        k0 = kk * _NBUF
        for b in range(_NBUF):
            k = k0 + b
            nb = (b + _LOOK) % _NBUF

            @pl.when(k >= _NBUF - _LOOK)
            def _():
                scat_wait(k - (_NBUF - _LOOK), nb)

            @pl.when(k + _LOOK < _K)
            def _():
                gath(k + _LOOK, nb)

            gath_wait(k, b)
            scat(k, b)
        return 0

    lax.fori_loop(0, _K // _NBUF, body, 0)
    for j in range(_K - (_NBUF - _LOOK), _K):
        scat_wait(j, j % _NBUF)
    plsc.subcore_barrier()

    pltpu.sync_copy(agg_sh.at[pl.ds(rbase, _RPT)],
                    out_hbm.at[pl.ds(cid * _NP + rbase, _RPT)])


def _tc_in_body(h_ref, w1_ref, y_ref):
    y_ref[...] = jnp.dot(h_ref[...], w1_ref[...],
                         preferred_element_type=jnp.float32)


def _tc_mid_body(y_ref, agg_ref, b1_ref, w2_ref, b2_ref, w1n_ref,
                 yn_ref, s_ref):
    z = y_ref[...] + agg_ref[0] + agg_ref[1] + b1_ref[...]
    z = jnp.maximum(z, 0.0)
    xn = jnp.dot(z, w2_ref[...], preferred_element_type=jnp.float32)
    xn = xn + b2_ref[...]
    mask = lax.broadcasted_iota(jnp.int32, xn.shape, 0) < _N
    xn = jnp.where(mask, xn, 0.0)
    yn_ref[...] = jnp.dot(xn, w1n_ref[...], preferred_element_type=jnp.float32)
    s_ref[...] = jnp.sum(xn, axis=0, keepdims=True) * (1.0 / _N)


def _tc_out_body(y_ref, agg_ref, b1_ref, w2_ref, b2_ref, s_ref):
    z = y_ref[...] + agg_ref[0] + agg_ref[1] + b1_ref[...]
    z = jnp.maximum(z, 0.0)
    xn = jnp.dot(z, w2_ref[...], preferred_element_type=jnp.float32)
    xn = xn + b2_ref[...]
    mask = lax.broadcasted_iota(jnp.int32, xn.shape, 0) < _N
    xn = jnp.where(mask, xn, 0.0)
    s_ref[...] = jnp.sum(xn, axis=0, keepdims=True) * (1.0 / _N)


_tc_in = pl.pallas_call(
    _tc_in_body,
    out_shape=jax.ShapeDtypeStruct((_NP, _DH), jnp.float32),
)

_tc_mid = pl.pallas_call(
    _tc_mid_body,
    out_shape=(
        jax.ShapeDtypeStruct((_NP, _DH), jnp.float32),
        jax.ShapeDtypeStruct((1, _DH), jnp.float32),
    ),
)

_tc_out = pl.pallas_call(
    _tc_out_body,
    out_shape=jax.ShapeDtypeStruct((1, _DH), jnp.float32),
)


def kernel(h, edge_index, W1_0, b1_0, W2_0, b2_0, W1_1, b1_1, W2_1, b2_1,
           W1_2, b1_2, W2_2, b2_2):
    src = edge_index[0]
    dst = edge_index[1]
    pad = _EPAD - _E
    srcp = jnp.concatenate(
        [src, jnp.full((pad,), _N, jnp.int32)]).reshape(_NW, _K, _BATCH)
    dstp = jnp.concatenate(
        [dst, jnp.zeros((pad,), jnp.int32)]).reshape(_NW, _K, _BATCH)
    h_pad = jnp.pad(h, ((0, _NP - _N), (0, 0)))
    zeros = jnp.zeros((_RPT, _DH), jnp.float32)

    y = _tc_in(h_pad, W1_0)

    agg = _sc_agg(y, srcp, dstp, zeros).reshape(_NC, _NP, _DH)
    y, s0 = _tc_mid(y, agg, b1_0.reshape(1, _DH), W2_0,
                    b2_0.reshape(1, _DH), W1_1)

    agg = _sc_agg(y, srcp, dstp, zeros).reshape(_NC, _NP, _DH)
    y, s1 = _tc_mid(y, agg, b1_1.reshape(1, _DH), W2_1,
                    b2_1.reshape(1, _DH), W1_2)

    agg = _sc_agg(y, srcp, dstp, zeros).reshape(_NC, _NP, _DH)
    s2 = _tc_out(y, agg, b1_2.reshape(1, _DH), W2_2, b2_2.reshape(1, _DH))

    return jnp.concatenate([s0[0], s1[0], s2[0]])


# speedup vs baseline: 2.3082x; 1.0285x over previous
import functools

import jax
import jax.numpy as jnp
from jax import lax
from jax.experimental import pallas as pl
from jax.experimental.pallas import tpu as pltpu
from jax.experimental.pallas import tpu_sc as plsc

_N = 10000
_E = 320000
_DH = 32
_NP = 10112

_NC = 2
_NS = 16
_NW = _NC * _NS
_BATCH = 128
_K = 80
_EPAD = _NW * _BATCH * _K
_RPT = _NP // _NS
_MROW = 1
_KS = _K // _MROW
_NBUF = 8
_LOOK = 4


_mesh = plsc.VectorSubcoreMesh(core_axis_name="c", subcore_axis_name="s")


@functools.partial(
    pl.kernel,
    out_type=jax.ShapeDtypeStruct((_NC * _NP, _DH), jnp.float32),
    mesh=_mesh,
    scratch_types=[
        pltpu.VMEM((_K, _BATCH), jnp.int32),
        pltpu.VMEM((_K, _BATCH), jnp.int32),
        pltpu.VMEM((_NBUF, _BATCH, _DH), jnp.float32),
        pltpu.VMEM_SHARED((_NP, _DH), jnp.float32),
        pltpu.VMEM_SHARED((_NP, _DH), jnp.float32),
        pltpu.SemaphoreType.DMA((_NBUF,)),
        pltpu.SemaphoreType.DMA((_NBUF,)),
    ],
    compiler_params=pltpu.CompilerParams(use_tc_tiling_on_sc=False),
)
def _sc_agg(y_hbm, src_hbm, dst_hbm, zeros_hbm, out_hbm,
            src_v, dst_v, rows_v, agg_sh, tbl_sh, gsem, ssem):
    cid = lax.axis_index("c")
    sid = lax.axis_index("s")
    wid = cid * _NS + sid
    rbase = sid * _RPT

    st0 = pltpu.async_copy(y_hbm.at[pl.ds(rbase, _RPT)],
                           tbl_sh.at[pl.ds(rbase, _RPT)], gsem.at[0])
    st1 = pltpu.async_copy(zeros_hbm, agg_sh.at[pl.ds(rbase, _RPT)],
                           gsem.at[1])
    st2 = pltpu.async_copy(src_hbm.at[wid], src_v, ssem.at[0])
    st3 = pltpu.async_copy(dst_hbm.at[wid], dst_v, ssem.at[1])
    st0.wait()
    st1.wait()
    st2.wait()
    st3.wait()
    plsc.subcore_barrier()

    def gath(k, b):
        pltpu.async_copy(tbl_sh.at[src_v.at[k]], rows_v.at[b], gsem.at[b])

    def gath_wait(k, b):
        pltpu.make_async_copy(tbl_sh.at[src_v.at[k]], rows_v.at[b],
                              gsem.at[b]).wait()

    def scat(k, b):
        pltpu.async_copy(rows_v.at[b], agg_sh.at[dst_v.at[k]],
                         ssem.at[b], add=True)

    def scat_wait(k, b):
        pltpu.make_async_copy(rows_v.at[b], agg_sh.at[dst_v.at[k]],
                              ssem.at[b]).wait()

    for j in range(_LOOK):
        gath(j, j)

    def body(kk, _):
        k0 = kk * _NBUF
        for b in range(_NBUF):
            k = k0 + b
            nb = (b + _LOOK) % _NBUF

            @pl.when(k >= _NBUF - _LOOK)
            def _():
                scat_wait(k - (_NBUF - _LOOK), nb)

            @pl.when(k + _LOOK < _KS)
            def _():
                gath(k + _LOOK, nb)

            gath_wait(k, b)
            scat(k, b)
        return 0

    lax.fori_loop(0, _KS // _NBUF, body, 0)
    for j in range(_KS - (_NBUF - _LOOK), _KS):
        scat_wait(j, j % _NBUF)
    plsc.subcore_barrier()

    pltpu.sync_copy(agg_sh.at[pl.ds(rbase, _RPT)],
                    out_hbm.at[pl.ds(cid * _NP + rbase, _RPT)])


def _tc_in_body(h_ref, w1_ref, y_ref):
    y_ref[...] = jnp.dot(h_ref[...], w1_ref[...],
                         preferred_element_type=jnp.float32)


def _tc_mid_body(y_ref, agg_ref, b1_ref, w2_ref, b2_ref, w1n_ref,
                 yn_ref, s_ref):
    z = y_ref[...] + agg_ref[0] + agg_ref[1] + b1_ref[...]
    z = jnp.maximum(z, 0.0)
    xn = jnp.dot(z, w2_ref[...], preferred_element_type=jnp.float32)
    xn = xn + b2_ref[...]
    mask = lax.broadcasted_iota(jnp.int32, xn.shape, 0) < _N
    xn = jnp.where(mask, xn, 0.0)
    yn_ref[...] = jnp.dot(xn, w1n_ref[...], preferred_element_type=jnp.float32)
    s_ref[...] = jnp.sum(xn, axis=0, keepdims=True) * (1.0 / _N)


def _tc_out_body(y_ref, agg_ref, b1_ref, w2_ref, b2_ref, s_ref):
    z = y_ref[...] + agg_ref[0] + agg_ref[1] + b1_ref[...]
    z = jnp.maximum(z, 0.0)
    xn = jnp.dot(z, w2_ref[...], preferred_element_type=jnp.float32)
    xn = xn + b2_ref[...]
    mask = lax.broadcasted_iota(jnp.int32, xn.shape, 0) < _N
    xn = jnp.where(mask, xn, 0.0)
    s_ref[...] = jnp.sum(xn, axis=0, keepdims=True) * (1.0 / _N)


_tc_in = pl.pallas_call(
    _tc_in_body,
    out_shape=jax.ShapeDtypeStruct((_NP, _DH), jnp.float32),
)

_tc_mid = pl.pallas_call(
    _tc_mid_body,
    out_shape=(
        jax.ShapeDtypeStruct((_NP, _DH), jnp.float32),
        jax.ShapeDtypeStruct((1, _DH), jnp.float32),
    ),
)

_tc_out = pl.pallas_call(
    _tc_out_body,
    out_shape=jax.ShapeDtypeStruct((1, _DH), jnp.float32),
)


def kernel(h, edge_index, W1_0, b1_0, W2_0, b2_0, W1_1, b1_1, W2_1, b2_1,
           W1_2, b1_2, W2_2, b2_2):
    src = edge_index[0]
    dst = edge_index[1]
    pad = _EPAD - _E
    srcp = jnp.concatenate(
        [src, jnp.full((pad,), _N, jnp.int32)]).reshape(_NW, _K, _BATCH)
    dstp = jnp.concatenate(
        [dst, jnp.zeros((pad,), jnp.int32)]).reshape(_NW, _K, _BATCH)
    h_pad = jnp.pad(h, ((0, _NP - _N), (0, 0)))
    zeros = jnp.zeros((_RPT, _DH), jnp.float32)

    y = _tc_in(h_pad, W1_0)

    agg = _sc_agg(y, srcp, dstp, zeros).reshape(_NC, _NP, _DH)
    y, s0 = _tc_mid(y, agg, b1_0.reshape(1, _DH), W2_0,
                    b2_0.reshape(1, _DH), W1_1)

    agg = _sc_agg(y, srcp, dstp, zeros).reshape(_NC, _NP, _DH)
    y, s1 = _tc_mid(y, agg, b1_1.reshape(1, _DH), W2_1,
                    b2_1.reshape(1, _DH), W1_2)

    agg = _sc_agg(y, srcp, dstp, zeros).reshape(_NC, _NP, _DH)
    s2 = _tc_out(y, agg, b1_2.reshape(1, _DH), W2_2, b2_2.reshape(1, _DH))

    return jnp.concatenate([s0[0], s1[0], s2[0]])


# speedup vs baseline: 2.9758x; 1.2893x over previous
import functools

import jax
import jax.numpy as jnp
from jax import lax
from jax.experimental import pallas as pl
from jax.experimental.pallas import tpu as pltpu
from jax.experimental.pallas import tpu_sc as plsc

_N = 10000
_E = 320000
_DH = 32
_NP = 10112

_NC = 2
_NS = 16
_NW = _NC * _NS
_BATCH = 128
_K = 80
_EPAD = _NW * _BATCH * _K
_RPT = _NP // _NS
_MROW = 1
_KS = _K // _MROW
_NBUF = 5
_LOOK = 3
_N4 = _NP // 4
_NB = _N // 4
_RP4 = _RPT // 4


_mesh = plsc.VectorSubcoreMesh(core_axis_name="c", subcore_axis_name="s")


@functools.partial(
    pl.kernel,
    out_type=jax.ShapeDtypeStruct((_NC * _N4, 128), jnp.float32),
    mesh=_mesh,
    scratch_types=[
        pltpu.VMEM((_K, _BATCH), jnp.int32),
        pltpu.VMEM((_K, _BATCH), jnp.int32),
        pltpu.VMEM((_NBUF, _BATCH, _DH), jnp.float32),
        pltpu.VMEM_SHARED((_NP, _DH), jnp.float32),
        pltpu.VMEM_SHARED((_NP, _DH), jnp.float32),
        pltpu.VMEM((_RPT, _DH), jnp.float32),
        pltpu.VMEM((_RP4, 128), jnp.float32),
        pltpu.SemaphoreType.DMA((_NBUF,)),
        pltpu.SemaphoreType.DMA((_NBUF,)),
    ],
    compiler_params=pltpu.CompilerParams(use_tc_tiling_on_sc=False),
)
def _sc_agg(y_hbm, ep_hbm, zeros_hbm, out_hbm,
            src_v, dst_v, rows_v, agg_sh, tbl_sh, t32, t4, gsem, ssem):
    cid = lax.axis_index("c")
    sid = lax.axis_index("s")
    wid = cid * _NS + sid
    rbase = sid * _RPT

    st0 = pltpu.async_copy(y_hbm.at[pl.ds(sid * _RP4, _RP4)], t4, gsem.at[0])
    st1 = pltpu.async_copy(zeros_hbm, agg_sh.at[pl.ds(rbase, _RPT)],
                           gsem.at[1])
    st2 = pltpu.async_copy(ep_hbm.at[wid], src_v, ssem.at[0])

    st0.wait()

    def rin(i, _):
        for c in range(4):
            r = i * 4 + c
            t32[r, pl.ds(0, 16)] = t4[i, pl.ds(32 * c, 16)]
            t32[r, pl.ds(16, 16)] = t4[i, pl.ds(32 * c + 16, 16)]
        return 0
    lax.fori_loop(0, _RP4, rin, 0)
    st3 = pltpu.async_copy(t32, tbl_sh.at[pl.ds(rbase, _RPT)], gsem.at[2])

    st2.wait()

    def up(i, _):
        for c in range(8):
            v = src_v[i, pl.ds(16 * c, 16)]
            src_v[i, pl.ds(16 * c, 16)] = v >> 16
            dst_v[i, pl.ds(16 * c, 16)] = v & 0xFFFF
        return 0
    lax.fori_loop(0, _K, up, 0)

    st1.wait()
    st3.wait()
    plsc.subcore_barrier()

    def gath(k, b):
        pltpu.async_copy(tbl_sh.at[src_v.at[k]], rows_v.at[b], gsem.at[b])

    def gath_wait(k, b):
        pltpu.make_async_copy(tbl_sh.at[src_v.at[k]], rows_v.at[b],
                              gsem.at[b]).wait()

    def scat(k, b):
        pltpu.async_copy(rows_v.at[b], agg_sh.at[dst_v.at[k]],
                         ssem.at[b], add=True)

    def scat_wait(k, b):
        pltpu.make_async_copy(rows_v.at[b], agg_sh.at[dst_v.at[k]],
                              ssem.at[b]).wait()

    for j in range(_LOOK):
        gath(j, j)

    def body(kk, _):
        k0 = kk * _NBUF
        for b in range(_NBUF):
            k = k0 + b
            nb = (b + _LOOK) % _NBUF

            @pl.when(k >= _NBUF - _LOOK)
            def _():
                scat_wait(k - (_NBUF - _LOOK), nb)

            @pl.when(k + _LOOK < _KS)
            def _():
                gath(k + _LOOK, nb)

            gath_wait(k, b)
            scat(k, b)
        return 0

    lax.fori_loop(0, _KS // _NBUF, body, 0)
    for j in range(_KS - (_NBUF - _LOOK), _KS):
        scat_wait(j, j % _NBUF)
    plsc.subcore_barrier()

    pltpu.sync_copy(agg_sh.at[pl.ds(rbase, _RPT)], t32)

    def rc(i, _):
        for c in range(4):
            r = i * 4 + c
            t4[i, pl.ds(32 * c, 16)] = t32[r, pl.ds(0, 16)]
            t4[i, pl.ds(32 * c + 16, 16)] = t32[r, pl.ds(16, 16)]
        return 0
    lax.fori_loop(0, _RP4, rc, 0)
    pltpu.sync_copy(t4, out_hbm.at[pl.ds(cid * _N4 + sid * _RP4, _RP4)])


def _blockdiag(w):
    wt = jnp.concatenate([jnp.concatenate([w] * 4, axis=1)] * 4, axis=0)
    ib = lax.broadcasted_iota(jnp.int32, (128, 128), 0) // _DH
    jb = lax.broadcasted_iota(jnp.int32, (128, 128), 1) // _DH
    return jnp.where(ib == jb, wt, 0.0)


def _tile4(v):
    return jnp.concatenate([v] * 4, axis=1)


def _tc_in_body(hp_ref, w1bd_ref, y_ref):
    y_ref[...] = jnp.dot(hp_ref[...], w1bd_ref[...],
                         preferred_element_type=jnp.float32)


def _relu_pack(y_ref, agg_ref, b1_ref):
    z = (y_ref[...] + agg_ref[pl.ds(0, _N4), :] + agg_ref[pl.ds(_N4, _N4), :]
         + _tile4(jnp.reshape(b1_ref[...], (1, _DH))))
    r = jnp.maximum(z, 0.0)
    mask = lax.broadcasted_iota(jnp.int32, (_N4, 128), 0) < _NB
    return jnp.where(mask, r, 0.0)


def _readout(r, w2_ref, b2_ref):
    srow = jnp.sum(r, axis=0, keepdims=True)
    s32 = (srow[:, 0:32] + srow[:, 32:64] + srow[:, 64:96] + srow[:, 96:128])
    s = jnp.dot(s32, w2_ref[...], preferred_element_type=jnp.float32)
    return s * (1.0 / _N) + jnp.reshape(b2_ref[...], (1, _DH))


def _tc_mid_body(y_ref, agg_ref, b1_ref, w2_ref, b2_ref, w1n_ref,
                 yn_ref, s_ref):
    r = _relu_pack(y_ref, agg_ref, b1_ref)
    wc = jnp.dot(w2_ref[...], w1n_ref[...], preferred_element_type=jnp.float32)
    bb = jnp.dot(jnp.reshape(b2_ref[...], (1, _DH)), w1n_ref[...],
                 preferred_element_type=jnp.float32)
    yn = jnp.dot(r, _blockdiag(wc), preferred_element_type=jnp.float32)
    yn = yn + _tile4(bb)
    mask = lax.broadcasted_iota(jnp.int32, (_N4, 128), 0) < _NB
    yn_ref[...] = jnp.where(mask, yn, 0.0)
    s_ref[...] = _readout(r, w2_ref, b2_ref)


def _tc_out_body(y_ref, agg_ref, b1_ref, w2_ref, b2_ref, s_ref):
    r = _relu_pack(y_ref, agg_ref, b1_ref)
    s_ref[...] = _readout(r, w2_ref, b2_ref)


_tc_in = pl.pallas_call(
    _tc_in_body,
    out_shape=jax.ShapeDtypeStruct((_N4, 128), jnp.float32),
)

_tc_mid = pl.pallas_call(
    _tc_mid_body,
    out_shape=(
        jax.ShapeDtypeStruct((_N4, 128), jnp.float32),
        jax.ShapeDtypeStruct((1, _DH), jnp.float32),
    ),
)

_tc_out = pl.pallas_call(
    _tc_out_body,
    out_shape=jax.ShapeDtypeStruct((1, _DH), jnp.float32),
)


def kernel(h, edge_index, W1_0, b1_0, W2_0, b2_0, W1_1, b1_1, W2_1, b2_1,
           W1_2, b1_2, W2_2, b2_2):
    pad = _EPAD - _E
    packed = edge_index[0] * 65536 + edge_index[1]
    ep = jnp.concatenate(
        [packed, jnp.full((pad,), _N * 65536, jnp.int32)]
    ).reshape(_NW, _K, _BATCH)
    zeros = jnp.zeros((_RPT, _DH), jnp.float32)

    hp = jnp.pad(h, ((0, _NP - _N), (0, 0))).reshape(_N4, 512)
    w1bd = jnp.kron(jnp.eye(4, dtype=jnp.float32), W1_0)

    y4 = _tc_in(hp, w1bd)

    agg4 = _sc_agg(y4, ep, zeros)
    y4, s0 = _tc_mid(y4, agg4, b1_0, W2_0, b2_0, W1_1)

    agg4 = _sc_agg(y4, ep, zeros)
    y4, s1 = _tc_mid(y4, agg4, b1_1, W2_1, b2_1, W1_2)

    agg4 = _sc_agg(y4, ep, zeros)
    s2 = _tc_out(y4, agg4, b1_2, W2_2, b2_2)

    return jnp.concatenate([s0[0], s1[0], s2[0]])


# speedup vs baseline: 3.1892x; 1.0717x over previous
import functools

import jax
import jax.numpy as jnp
from jax import lax
from jax.experimental import pallas as pl
from jax.experimental.pallas import tpu as pltpu
from jax.experimental.pallas import tpu_sc as plsc

_N = 10000
_E = 320000
_DH = 32
_NP = 10112

_NC = 2
_NS = 16
_NW = _NC * _NS
_BATCH = 128
_K = 80
_EPAD = _NW * _BATCH * _K
_RPT = _NP // _NS
_MROW = 1
_KS = _K // _MROW
_NBUF = 5
_LOOK = 3
_N4 = _NP // 4
_NB = _N // 4
_RP4 = _RPT // 4


_mesh = plsc.VectorSubcoreMesh(core_axis_name="c", subcore_axis_name="s")


@functools.partial(
    pl.kernel,
    out_type=jax.ShapeDtypeStruct((_NC * _N4, 128), jnp.float32),
    mesh=_mesh,
    scratch_types=[
        pltpu.VMEM((_K, _BATCH), jnp.int32),
        pltpu.VMEM((_K, _BATCH), jnp.int32),
        pltpu.VMEM((_NBUF, _BATCH, _DH), jnp.float32),
        pltpu.VMEM_SHARED((_NP, _DH), jnp.float32),
        pltpu.VMEM_SHARED((_NP, _DH), jnp.float32),
        pltpu.VMEM((_RPT, _DH), jnp.float32),
        pltpu.VMEM((_RP4, 128), jnp.float32),
        pltpu.SemaphoreType.DMA((_NBUF,)),
        pltpu.SemaphoreType.DMA((_NBUF,)),
    ],
    compiler_params=pltpu.CompilerParams(use_tc_tiling_on_sc=False),
)
def _sc_agg(y_hbm, ep_hbm, zeros_hbm, out_hbm,
            src_v, dst_v, rows_v, agg_sh, tbl_sh, t32, t4, gsem, ssem):
    cid = lax.axis_index("c")
    sid = lax.axis_index("s")
    wid = cid * _NS + sid
    rbase = sid * _RPT

    st0 = pltpu.async_copy(y_hbm.at[pl.ds(sid * _RP4, _RP4)], t4, gsem.at[0])
    st1 = pltpu.async_copy(zeros_hbm, agg_sh.at[pl.ds(rbase, _RPT)],
                           gsem.at[1])
    st2 = pltpu.async_copy(ep_hbm.at[pl.ds(wid * _K, _K)], src_v,
                           ssem.at[0])

    st0.wait()

    def rin(i, _):
        for c in range(4):
            r = i * 4 + c
            t32[r, pl.ds(0, 16)] = t4[i, pl.ds(32 * c, 16)]
            t32[r, pl.ds(16, 16)] = t4[i, pl.ds(32 * c + 16, 16)]
        return 0
    lax.fori_loop(0, _RP4, rin, 0)
    st3 = pltpu.async_copy(t32, tbl_sh.at[pl.ds(rbase, _RPT)], gsem.at[2])

    st2.wait()

    def up(i, _):
        for c in range(8):
            v = src_v[i, pl.ds(16 * c, 16)]
            src_v[i, pl.ds(16 * c, 16)] = v >> 16
            dst_v[i, pl.ds(16 * c, 16)] = v & 0xFFFF
        return 0
    lax.fori_loop(0, _K, up, 0)

    st1.wait()
    st3.wait()
    plsc.subcore_barrier()

    def gath(k, b):
        pltpu.async_copy(tbl_sh.at[src_v.at[k]], rows_v.at[b], gsem.at[b])

    def gath_wait(k, b):
        pltpu.make_async_copy(tbl_sh.at[src_v.at[k]], rows_v.at[b],
                              gsem.at[b]).wait()

    def scat(k, b):
        pltpu.async_copy(rows_v.at[b], agg_sh.at[dst_v.at[k]],
                         ssem.at[b], add=True)

    def scat_wait(k, b):
        pltpu.make_async_copy(rows_v.at[b], agg_sh.at[dst_v.at[k]],
                              ssem.at[b]).wait()

    for j in range(_LOOK):
        gath(j, j)

    def body(kk, _):
        k0 = kk * _NBUF
        for b in range(_NBUF):
            k = k0 + b
            nb = (b + _LOOK) % _NBUF

            @pl.when(k >= _NBUF - _LOOK)
            def _():
                scat_wait(k - (_NBUF - _LOOK), nb)

            @pl.when(k + _LOOK < _KS)
            def _():
                gath(k + _LOOK, nb)

            gath_wait(k, b)
            scat(k, b)
        return 0

    lax.fori_loop(0, _KS // _NBUF, body, 0)
    for j in range(_KS - (_NBUF - _LOOK), _KS):
        scat_wait(j, j % _NBUF)
    plsc.subcore_barrier()

    pltpu.sync_copy(agg_sh.at[pl.ds(rbase, _RPT)], t32)

    def rc(i, _):
        for c in range(4):
            r = i * 4 + c
            t4[i, pl.ds(32 * c, 16)] = t32[r, pl.ds(0, 16)]
            t4[i, pl.ds(32 * c + 16, 16)] = t32[r, pl.ds(16, 16)]
        return 0
    lax.fori_loop(0, _RP4, rc, 0)
    pltpu.sync_copy(t4, out_hbm.at[pl.ds(cid * _N4 + sid * _RP4, _RP4)])


def _blockdiag(w):
    wt = jnp.concatenate([jnp.concatenate([w] * 4, axis=1)] * 4, axis=0)
    ib = lax.broadcasted_iota(jnp.int32, (128, 128), 0) // _DH
    jb = lax.broadcasted_iota(jnp.int32, (128, 128), 1) // _DH
    return jnp.where(ib == jb, wt, 0.0)


def _tile4(v):
    return jnp.concatenate([v] * 4, axis=1)


def _tc_in_body(hp_ref, w1_ref, y_ref):
    w1 = w1_ref[...]
    wt = jnp.concatenate([jnp.concatenate([w1] * 4, axis=1)] * 4, axis=0)
    ib = lax.broadcasted_iota(jnp.int32, (512, 128), 0) // 128
    jb = lax.broadcasted_iota(jnp.int32, (512, 128), 1) // _DH
    w1bd = jnp.where(ib == jb, wt, 0.0)
    y_ref[...] = jnp.dot(hp_ref[...], w1bd,
                         preferred_element_type=jnp.float32)


def _tc_pack_body(ei_ref, ep_ref):
    p = ei_ref[0, :] * 65536 + ei_ref[1, :]
    ep_ref[pl.ds(0, _E // 128), :] = jnp.reshape(p, (_E // 128, 128))
    ep_ref[pl.ds(_E // 128, (_EPAD - _E) // 128), :] = jnp.full(
        ((_EPAD - _E) // 128, 128), _N * 65536, jnp.int32)


_tc_pack = pl.pallas_call(
    _tc_pack_body,
    out_shape=jax.ShapeDtypeStruct((_EPAD // 128, 128), jnp.int32),
)


def _relu_pack(y_ref, agg_ref, b1_ref):
    z = (y_ref[...] + agg_ref[pl.ds(0, _N4), :] + agg_ref[pl.ds(_N4, _N4), :]
         + _tile4(jnp.reshape(b1_ref[...], (1, _DH))))
    r = jnp.maximum(z, 0.0)
    mask = lax.broadcasted_iota(jnp.int32, (_N4, 128), 0) < _NB
    return jnp.where(mask, r, 0.0)


def _readout(r, w2_ref, b2_ref):
    srow = jnp.sum(r, axis=0, keepdims=True)
    s32 = (srow[:, 0:32] + srow[:, 32:64] + srow[:, 64:96] + srow[:, 96:128])
    s = jnp.dot(s32, w2_ref[...], preferred_element_type=jnp.float32)
    return s * (1.0 / _N) + jnp.reshape(b2_ref[...], (1, _DH))


def _tc_mid_body(y_ref, agg_ref, b1_ref, w2_ref, b2_ref, w1n_ref,
                 yn_ref, s_ref):
    r = _relu_pack(y_ref, agg_ref, b1_ref)
    wc = jnp.dot(w2_ref[...], w1n_ref[...], preferred_element_type=jnp.float32)
    bb = jnp.dot(jnp.reshape(b2_ref[...], (1, _DH)), w1n_ref[...],
                 preferred_element_type=jnp.float32)
    yn = jnp.dot(r, _blockdiag(wc), preferred_element_type=jnp.float32)
    yn = yn + _tile4(bb)
    mask = lax.broadcasted_iota(jnp.int32, (_N4, 128), 0) < _NB
    yn_ref[...] = jnp.where(mask, yn, 0.0)
    s_ref[...] = _readout(r, w2_ref, b2_ref)


def _tc_out_body(y_ref, agg_ref, b1_ref, w2_ref, b2_ref, s_ref):
    r = _relu_pack(y_ref, agg_ref, b1_ref)
    s_ref[...] = _readout(r, w2_ref, b2_ref)


_tc_in = pl.pallas_call(
    _tc_in_body,
    out_shape=jax.ShapeDtypeStruct((_N4, 128), jnp.float32),
)

_tc_mid = pl.pallas_call(
    _tc_mid_body,
    out_shape=(
        jax.ShapeDtypeStruct((_N4, 128), jnp.float32),
        jax.ShapeDtypeStruct((1, _DH), jnp.float32),
    ),
)

_tc_out = pl.pallas_call(
    _tc_out_body,
    out_shape=jax.ShapeDtypeStruct((1, _DH), jnp.float32),
)


def kernel(h, edge_index, W1_0, b1_0, W2_0, b2_0, W1_1, b1_1, W2_1, b2_1,
           W1_2, b1_2, W2_2, b2_2):
    ep = _tc_pack(edge_index)
    zeros = jnp.zeros((_RPT, _DH), jnp.float32)

    hp = jnp.pad(h, ((0, _NP - _N), (0, 0))).reshape(_N4, 512)

    y4 = _tc_in(hp, W1_0)

    agg4 = _sc_agg(y4, ep, zeros)
    y4, s0 = _tc_mid(y4, agg4, b1_0, W2_0, b2_0, W1_1)

    agg4 = _sc_agg(y4, ep, zeros)
    y4, s1 = _tc_mid(y4, agg4, b1_1, W2_1, b2_1, W1_2)

    agg4 = _sc_agg(y4, ep, zeros)
    s2 = _tc_out(y4, agg4, b1_2, W2_2, b2_2)

    return jnp.concatenate([s0[0], s1[0], s2[0]])
